# baseline (speedup 1.0000x reference)
import functools

import jax
import jax.numpy as jnp
from jax import lax
from jax.experimental import pallas as pl
from jax.experimental.pallas import tpu as pltpu
from jax.experimental.pallas import tpu_sc as plsc

_N = 10000
_E = 320000
_D = 128

_NC = 2
_NS = 16
_NW = _NC * _NS
_EW = _E // _NW
_C = 80
_NCH = _EW // _C
_NB = 3
_RS = 624
_TAIL = _N - _NS * _RS
_ZR = 104


def _sc_segment_partials(col, edge_attr):
    mesh = plsc.VectorSubcoreMesh(core_axis_name="c", subcore_axis_name="s")

    @functools.partial(
        pl.kernel,
        mesh=mesh,
        out_type=jax.ShapeDtypeStruct((_NC, _N, _D), jnp.float32),
        scratch_types=[
            pltpu.VMEM((_NB, _C), jnp.int32),
            pltpu.VMEM((_NB, _C, _D), jnp.float32),
            pltpu.VMEM((_ZR, _D), jnp.float32),
            pltpu.VMEM_SHARED((_N, _D), jnp.float32),
            pltpu.SemaphoreType.DMA((_NB,)),
            pltpu.SemaphoreType.DMA((_NB,)),
            pltpu.SemaphoreType.DMA((_NB,)),
        ],
    )
    def k(col_hbm, ea_hbm, out_hbm, idx_v, ebuf, zbuf, agg_sh, sem_i, sem_d,
          sem_s):
        cid = lax.axis_index("c")
        sid = lax.axis_index("s")
        wid = sid * _NC + cid
        ebase = wid * _EW

        def start_load(ch, b):
            off = ebase + ch * _C
            pltpu.make_async_copy(col_hbm.at[pl.ds(off, _C)],
                                  idx_v.at[b], sem_i.at[b]).start()
            pltpu.make_async_copy(ea_hbm.at[pl.ds(off, _C)],
                                  ebuf.at[b], sem_d.at[b]).start()

        def wait_load(b):
            pltpu.make_async_copy(col_hbm.at[pl.ds(0, _C)],
                                  idx_v.at[b], sem_i.at[b]).wait()
            pltpu.make_async_copy(ea_hbm.at[pl.ds(0, _C)],
                                  ebuf.at[b], sem_d.at[b]).wait()

        def fire_scatter(b):
            pltpu.async_copy(ebuf.at[b], agg_sh.at[idx_v.at[b]],
                             sem_s.at[b], add=True)

        def drain_scatter(b):
            pltpu.make_async_copy(ebuf.at[b], agg_sh.at[idx_v.at[b]],
                                  sem_s.at[b]).wait()

        start_load(0, 0)
        start_load(1, 1)

        def zrow(i, carry):
            for j in range(_D // 16):
                zbuf[i, pl.ds(j * 16, 16)] = jnp.zeros((16,), jnp.float32)
            return carry

        lax.fori_loop(0, _ZR, zrow, 0)
        for r in range(_RS // _ZR):
            pltpu.sync_copy(zbuf,
                            agg_sh.at[pl.ds(sid * _RS + r * _ZR, _ZR)])

        @pl.when(sid == _NS - 1)
        def _zero_tail():
            pltpu.sync_copy(zbuf.at[pl.ds(0, _TAIL)],
                            agg_sh.at[pl.ds(_NS * _RS, _TAIL)])

        plsc.subcore_barrier()

        wait_load(0)
        fire_scatter(0)
        start_load(2, 2)

        def body(i, carry):
            for q in range(3):
                t = 3 * i + 1 + q
                bt = (1 + q) % 3
                wait_load(bt)
                fire_scatter(bt)
                drain_scatter(q)

                @pl.when(t + 2 < _NCH)
                def _next():
                    start_load(t + 2, q)
            return carry

        lax.fori_loop(0, (_NCH - 2) // 3, body, 0)
        wait_load(1)
        fire_scatter(1)
        drain_scatter(0)
        drain_scatter(1)
        plsc.subcore_barrier()
        pltpu.sync_copy(agg_sh.at[pl.ds(sid * _RS, _RS)],
                        out_hbm.at[cid, pl.ds(sid * _RS, _RS)])

        @pl.when(sid == _NS - 1)
        def _write_tail():
            pltpu.sync_copy(agg_sh.at[pl.ds(_NS * _RS, _TAIL)],
                            out_hbm.at[cid, pl.ds(_NS * _RS, _TAIL)])

    return k(col, edge_attr)


def _ln(h, g, b):
    mu = jnp.mean(h, axis=-1, keepdims=True)
    d = h - mu
    var = jnp.mean(d * d, axis=-1, keepdims=True)
    return d * lax.rsqrt(var + 1e-5) * g + b


def _ssp(h):
    return jnp.maximum(h, 0.0) + jnp.log1p(jnp.exp(-jnp.abs(h))) - 0.6931471805599453


def _mlp_body(x_ref, p_ref, w1_ref, b1_ref, g1_ref, be1_ref,
              w2_ref, b2_ref, g2_ref, be2_ref,
              w3_ref, b3_ref, g3_ref, be3_ref, out_ref):
    agg = p_ref[0] + p_ref[1]
    cat = jnp.concatenate([x_ref[...], agg], axis=1)
    h = (jnp.dot(cat, w1_ref[...], preferred_element_type=jnp.float32)
         + b1_ref[...])
    h = _ssp(_ln(h, g1_ref[...], be1_ref[...]))
    h = jnp.dot(h, w2_ref[...], preferred_element_type=jnp.float32) + b2_ref[...]
    h = _ssp(_ln(h, g2_ref[...], be2_ref[...]))
    h = jnp.dot(h, w3_ref[...], preferred_element_type=jnp.float32) + b3_ref[...]
    h = _ssp(_ln(h, g3_ref[...], be3_ref[...]))
    out_ref[...] = h


def _mlp(x, partials, w1, b1, g1, be1, w2, b2, g2, be2, w3, b3, g3, be3):
    bn = 2000
    grid = _N // bn
    full = lambda i: (0, 0)
    vec = pl.BlockSpec((1, _D), full)
    mat = pl.BlockSpec((_D, _D), full)
    return pl.pallas_call(
        _mlp_body,
        grid=(grid,),
        in_specs=[
            pl.BlockSpec((bn, _D), lambda i: (i, 0)),
            pl.BlockSpec((_NC, bn, _D), lambda i: (0, i, 0)),
            pl.BlockSpec((2 * _D, _D), full), vec, vec, vec,
            mat, vec, vec, vec,
            mat, vec, vec, vec,
        ],
        out_specs=pl.BlockSpec((bn, _D), lambda i: (i, 0)),
        out_shape=jax.ShapeDtypeStruct((_N, _D), jnp.float32),
    )(x, partials, w1, b1, g1, be1, w2, b2, g2, be2, w3, b3, g3, be3)


def kernel(x, edge_index, edge_attr, W1, b1, g1, be1, W2, b2, g2, be2, W3, b3, g3, be3):
    partials = _sc_segment_partials(edge_index[1], edge_attr)
    r = lambda v: v.reshape(1, _D)
    return _mlp(x, partials, W1, r(b1), r(g1), r(be1),
                W2, r(b2), r(g2), r(be2), W3, r(b3), r(g3), r(be3))


# speedup vs baseline: 1.0795x; 1.0795x over previous
import functools

import jax
import jax.numpy as jnp
from jax import lax
from jax.experimental import pallas as pl
from jax.experimental.pallas import tpu as pltpu
from jax.experimental.pallas import tpu_sc as plsc

_N = 10000
_E = 320000
_D = 128

_NC = 2
_NS = 16
_NW = _NC * _NS
_EW = _E // _NW
_C = 80
_NCH = _EW // _C
_NB = 3
_RS = 624
_TAIL = _N - _NS * _RS
_ZR = 104


def _sc_segment_partials(ei_flat, edge_attr):
    mesh = plsc.VectorSubcoreMesh(core_axis_name="c", subcore_axis_name="s")

    @functools.partial(
        pl.kernel,
        mesh=mesh,
        out_type=jax.ShapeDtypeStruct((_NC, _N, _D), jnp.float32),
        scratch_types=[
            pltpu.VMEM((_NB, _C), jnp.int32),
            pltpu.VMEM((_NB, _C, _D), jnp.float32),
            pltpu.VMEM((_ZR, _D), jnp.float32),
            pltpu.VMEM_SHARED((_N, _D), jnp.float32),
            pltpu.SemaphoreType.DMA((_NB,)),
            pltpu.SemaphoreType.DMA((_NB,)),
            pltpu.SemaphoreType.DMA((_NB,)),
        ],
    )
    def k(col_hbm, ea_hbm, out_hbm, idx_v, ebuf, zbuf, agg_sh, sem_i, sem_d,
          sem_s):
        cid = lax.axis_index("c")
        sid = lax.axis_index("s")
        wid = sid * _NC + cid
        ebase = _E + wid * _EW

        def start_load(ch, b):
            off = ebase + ch * _C
            pltpu.make_async_copy(col_hbm.at[pl.ds(off, _C)],
                                  idx_v.at[b], sem_i.at[b]).start()
            pltpu.make_async_copy(ea_hbm.at[pl.ds(off - _E, _C)],
                                  ebuf.at[b], sem_d.at[b]).start()

        def wait_load(b):
            pltpu.make_async_copy(col_hbm.at[pl.ds(0, _C)],
                                  idx_v.at[b], sem_i.at[b]).wait()
            pltpu.make_async_copy(ea_hbm.at[pl.ds(0, _C)],
                                  ebuf.at[b], sem_d.at[b]).wait()

        def fire_scatter(b):
            pltpu.async_copy(ebuf.at[b], agg_sh.at[idx_v.at[b]],
                             sem_s.at[b], add=True)

        def drain_scatter(b):
            pltpu.make_async_copy(ebuf.at[b], agg_sh.at[idx_v.at[b]],
                                  sem_s.at[b]).wait()

        start_load(0, 0)
        start_load(1, 1)

        def zrow(i, carry):
            for j in range(_D // 16):
                zbuf[i, pl.ds(j * 16, 16)] = jnp.zeros((16,), jnp.float32)
            return carry

        lax.fori_loop(0, _ZR, zrow, 0)
        for r in range(_RS // _ZR):
            pltpu.sync_copy(zbuf,
                            agg_sh.at[pl.ds(sid * _RS + r * _ZR, _ZR)])

        @pl.when(sid == _NS - 1)
        def _zero_tail():
            pltpu.sync_copy(zbuf.at[pl.ds(0, _TAIL)],
                            agg_sh.at[pl.ds(_NS * _RS, _TAIL)])

        plsc.subcore_barrier()

        wait_load(0)
        fire_scatter(0)
        start_load(2, 2)

        def body(i, carry):
            for q in range(3):
                t = 3 * i + 1 + q
                bt = (1 + q) % 3
                wait_load(bt)
                fire_scatter(bt)
                drain_scatter(q)

                @pl.when(t + 2 < _NCH)
                def _next():
                    start_load(t + 2, q)
            return carry

        lax.fori_loop(0, (_NCH - 2) // 3, body, 0)
        wait_load(1)
        fire_scatter(1)
        drain_scatter(0)
        drain_scatter(1)
        plsc.subcore_barrier()
        pltpu.sync_copy(agg_sh.at[pl.ds(sid * _RS, _RS)],
                        out_hbm.at[cid, pl.ds(sid * _RS, _RS)])

        @pl.when(sid == _NS - 1)
        def _write_tail():
            pltpu.sync_copy(agg_sh.at[pl.ds(_NS * _RS, _TAIL)],
                            out_hbm.at[cid, pl.ds(_NS * _RS, _TAIL)])

    return k(ei_flat, edge_attr)


def _ln(h, g, b):
    mu = jnp.mean(h, axis=-1, keepdims=True)
    d = h - mu
    var = jnp.mean(d * d, axis=-1, keepdims=True)
    return d * lax.rsqrt(var + 1e-5) * g + b


def _ssp(h):
    return jnp.maximum(h, 0.0) + jnp.log1p(jnp.exp(-jnp.abs(h))) - 0.6931471805599453


def _mlp_body(x_ref, p_ref, w1_ref, b1_ref, g1_ref, be1_ref,
              w2_ref, b2_ref, g2_ref, be2_ref,
              w3_ref, b3_ref, g3_ref, be3_ref, out_ref):
    agg = p_ref[0] + p_ref[1]
    cat = jnp.concatenate([x_ref[...], agg], axis=1)
    h = (jnp.dot(cat, w1_ref[...], preferred_element_type=jnp.float32)
         + b1_ref[...])
    h = _ssp(_ln(h, g1_ref[...], be1_ref[...]))
    h = jnp.dot(h, w2_ref[...], preferred_element_type=jnp.float32) + b2_ref[...]
    h = _ssp(_ln(h, g2_ref[...], be2_ref[...]))
    h = jnp.dot(h, w3_ref[...], preferred_element_type=jnp.float32) + b3_ref[...]
    h = _ssp(_ln(h, g3_ref[...], be3_ref[...]))
    out_ref[...] = h


def _mlp(x, partials, w1, b1, g1, be1, w2, b2, g2, be2, w3, b3, g3, be3):
    bn = 2000
    grid = _N // bn
    full = lambda i: (0, 0)
    vec = pl.BlockSpec((1, _D), full)
    mat = pl.BlockSpec((_D, _D), full)
    return pl.pallas_call(
        _mlp_body,
        grid=(grid,),
        in_specs=[
            pl.BlockSpec((bn, _D), lambda i: (i, 0)),
            pl.BlockSpec((_NC, bn, _D), lambda i: (0, i, 0)),
            pl.BlockSpec((2 * _D, _D), full), vec, vec, vec,
            mat, vec, vec, vec,
            mat, vec, vec, vec,
        ],
        out_specs=pl.BlockSpec((bn, _D), lambda i: (i, 0)),
        out_shape=jax.ShapeDtypeStruct((_N, _D), jnp.float32),
    )(x, partials, w1, b1, g1, be1, w2, b2, g2, be2, w3, b3, g3, be3)


def kernel(x, edge_index, edge_attr, W1, b1, g1, be1, W2, b2, g2, be2, W3, b3, g3, be3):
    partials = _sc_segment_partials(edge_index.reshape(-1), edge_attr)
    r = lambda v: v.reshape(1, _D)
    return _mlp(x, partials, W1, r(b1), r(g1), r(be1),
                W2, r(b2), r(g2), r(be2), W3, r(b3), r(g3), r(be3))
